# Initial kernel scaffold; baseline (speedup 1.0000x reference)
import jax
import jax.numpy as jnp
from jax.experimental import pallas as pl


def kernel(x, W1, b1, gp, bp, W2, b2, A1w, A1b, A2w, A2b, A3w, A3b, C1w, C1b, g1, be1, C2w, C2b, g2, be2, C3w, C3b, g3, be3, Rw, Rb):
    raise NotImplementedError("write your pallas kernel here")


# speedup vs baseline: 916.2089x; 916.2089x over previous
import jax
import jax.numpy as jnp
from jax.experimental import pallas as pl
from jax.experimental.pallas import tpu as pltpu

_N, _IN, _H, _OUT = 512, 2048, 512, 128
_F32 = jnp.float32


def _dot(a, b):
    return jax.lax.dot_general(a, b, (((1,), (0,)), ((), ())),
                               preferred_element_type=_F32)


def _ln(h, g, b):
    mu = jnp.mean(h, axis=-1, keepdims=True)
    v = jnp.mean((h - mu) ** 2, axis=-1, keepdims=True)
    return (h - mu) / jnp.sqrt(v + 1e-5) * g + b


def _ncb_kernel(x_ref, W1_ref, b1_ref, gp_ref, bp_ref, W2_ref, b2_ref,
                A1w_ref, A1b_ref, A2w_ref, A2b_ref, A3w_ref, A3b_ref,
                C1w_ref, C1b_ref, g1_ref, be1_ref, C2w_ref, C2b_ref,
                g2_ref, be2_ref, C3w_ref, C3b_ref, g3_ref, be3_ref,
                Rw_ref, Rb_ref, h3_ref, att_ref, mam_ref):
    x = x_ref[...]
    h = jnp.maximum(_dot(x, W1_ref[...]) + b1_ref[...], 0.0)
    h = _ln(h, gp_ref[...], bp_ref[...])
    xp = _dot(h, W2_ref[...]) + b2_ref[...]
    a1 = jax.nn.sigmoid(_dot(xp, A1w_ref[...]) + A1b_ref[...])
    a2 = jnp.tanh(_dot(xp, A2w_ref[...]) + A2b_ref[...])
    att = _dot(a1 * a2, A3w_ref[...]) + A3b_ref[...]
    att_ref[...] = att
    s = jax.nn.sigmoid(att)
    mam_ref[...] = jax.lax.dot_general(
        s, s, (((1,), (1,)), ((), ())), preferred_element_type=_F32)
    deg = s * jnp.sum(s)
    a = s * jnp.where(deg > 0, jax.lax.rsqrt(deg), 0.0)

    def gcn(z, w_ref, b_ref):
        t = jax.lax.dot_general(a, z, (((0,), (0,)), ((), ())),
                                preferred_element_type=_F32)
        v = _dot(t, w_ref[...])
        return a * v + b_ref[...]

    h1 = _ln(jnp.maximum(gcn(xp, C1w_ref, C1b_ref), 0.0),
             g1_ref[...], be1_ref[...]) + xp
    h2 = _ln(jnp.maximum(gcn(h1, C2w_ref, C2b_ref), 0.0),
             g2_ref[...], be2_ref[...]) + h1
    h3_ref[...] = (_ln(jnp.maximum(gcn(h2, C3w_ref, C3b_ref), 0.0),
                       g3_ref[...], be3_ref[...])
                   + _dot(h2, Rw_ref[...]) + Rb_ref[...])


def _build(interpret=False):
    return pl.pallas_call(
        _ncb_kernel,
        out_shape=(
            jax.ShapeDtypeStruct((_N, _OUT), _F32),
            jax.ShapeDtypeStruct((_N, 1), _F32),
            jax.ShapeDtypeStruct((_N, _N), _F32),
        ),
        compiler_params=pltpu.CompilerParams(
            vmem_limit_bytes=110 * 1024 * 1024),
        interpret=interpret,
    )


def kernel(x, W1, b1, gp, bp, W2, b2, A1w, A1b, A2w, A2b, A3w, A3b,
           C1w, C1b, g1, be1, C2w, C2b, g2, be2, C3w, C3b, g3, be3, Rw, Rb):
    r = lambda v: v.reshape(1, -1)
    return _build()(
        x, W1, r(b1), r(gp), r(bp), W2, r(b2), A1w, r(A1b), A2w, r(A2b),
        A3w, r(A3b), C1w, r(C1b), r(g1), r(be1), C2w, r(C2b), r(g2), r(be2),
        C3w, r(C3b), r(g3), r(be3), Rw, r(Rb))
